# Initial kernel scaffold; baseline (speedup 1.0000x reference)
import jax
import jax.numpy as jnp
from jax.experimental import pallas as pl


def kernel(x, neigh_x, W_l, b_l, W_r, b_r):
    raise NotImplementedError("write your pallas kernel here")


# speedup vs baseline: 1.2187x; 1.2187x over previous
import jax
import jax.numpy as jnp
from jax.experimental import pallas as pl

N = 10000
K = 32
D = 128
BLOCK = 400


def _body(x_ref, n_ref, wl_ref, wr_ref, b_ref, o_ref):
    mean = jnp.mean(n_ref[...], axis=1)
    acc = jnp.dot(x_ref[...], wl_ref[...], preferred_element_type=jnp.float32)
    acc = acc + jnp.dot(mean, wr_ref[...], preferred_element_type=jnp.float32)
    o_ref[...] = acc + b_ref[...]


def kernel(x, neigh_x, W_l, b_l, W_r, b_r):
    wl_t = W_l.T
    wr_t = W_r.T
    bias = (b_l + b_r).reshape(1, D)
    grid = (N // BLOCK,)
    return pl.pallas_call(
        _body,
        grid=grid,
        in_specs=[
            pl.BlockSpec((BLOCK, D), lambda i: (i, 0)),
            pl.BlockSpec((BLOCK, K, D), lambda i: (i, 0, 0)),
            pl.BlockSpec((D, D), lambda i: (0, 0)),
            pl.BlockSpec((D, D), lambda i: (0, 0)),
            pl.BlockSpec((1, D), lambda i: (0, 0)),
        ],
        out_specs=pl.BlockSpec((BLOCK, D), lambda i: (i, 0)),
        out_shape=jax.ShapeDtypeStruct((N, D), jnp.float32),
    )(x, neigh_x, wl_t, wr_t, bias)
